# Initial kernel scaffold; baseline (speedup 1.0000x reference)
import jax
import jax.numpy as jnp
from jax.experimental import pallas as pl


def kernel(input_ids, token_table, pos_table):
    raise NotImplementedError("write your pallas kernel here")


# speedup vs baseline: 1.0090x; 1.0090x over previous
import functools

import jax
import jax.numpy as jnp
from jax import lax
from jax.experimental import pallas as pl
from jax.experimental.pallas import tpu as pltpu
from jax.experimental.pallas import tpu_sc as plsc

NC = 2
NS = 16
NW = NC * NS
LANES = 16


def _emb_body(s_per_w, rows_chunk, n_chunks, d,
              ids_hbm, tok_hbm, pos_hbm, out_hbm,
              idx_v, tok_v, pos_v, sem):
    b_per_w = rows_chunk * n_chunks
    wid = lax.axis_index("s") * NC + lax.axis_index("c")
    base = wid * b_per_w
    pos_base = (wid % (s_per_w)) * b_per_w

    for k in range(n_chunks):
        row0 = base + k * rows_chunk
        pltpu.sync_copy(ids_hbm.at[pl.ds(row0, rows_chunk)], idx_v.at[k])
        gather = pltpu.async_copy(tok_hbm.at[idx_v.at[k]], tok_v, sem)
        pltpu.sync_copy(
            pos_hbm.at[pl.ds(pos_base + k * rows_chunk, rows_chunk)], pos_v)
        gather.wait()

        def row_body(r, carry):
            for c in range(d // LANES):
                sl = pl.ds(c * LANES, LANES)
                plsc.addupdate(tok_v.at[r, sl], pos_v[r, sl])
            return carry

        lax.fori_loop(0, rows_chunk, row_body, 0)
        pltpu.sync_copy(tok_v, out_hbm.at[pl.ds(row0, rows_chunk)])


def kernel(input_ids, token_table, pos_table):
    batch, seq = input_ids.shape
    vocab, d = token_table.shape
    n = batch * seq
    ids_flat = input_ids.reshape(n).astype(jnp.int32)

    b_per_w = n // NW
    n_chunks = 4
    rows_chunk = b_per_w // n_chunks
    s_per_w = seq // b_per_w

    mesh = plsc.VectorSubcoreMesh(core_axis_name="c", subcore_axis_name="s")

    run = functools.partial(
        pl.kernel,
        mesh=mesh,
        out_type=jax.ShapeDtypeStruct((n, d), jnp.float32),
        scratch_types=[
            pltpu.VMEM((n_chunks, rows_chunk), jnp.int32),
            pltpu.VMEM((rows_chunk, d), jnp.float32),
            pltpu.VMEM((rows_chunk, d), jnp.float32),
            pltpu.SemaphoreType.DMA,
        ],
    )(functools.partial(_emb_body, s_per_w, rows_chunk, n_chunks, d))

    out = run(ids_flat, token_table, pos_table)
    return out.reshape(batch, seq, d)
